# Initial kernel scaffold; baseline (speedup 1.0000x reference)
import jax
import jax.numpy as jnp
from jax.experimental import pallas as pl


def kernel(h, edge_index, Wq, bq, Wk, bk, Wv, bv):
    raise NotImplementedError("write your pallas kernel here")


# speedup vs baseline: 15.7442x; 15.7442x over previous
import functools

import jax
import jax.numpy as jnp
from jax import lax
from jax.experimental import pallas as pl
from jax.experimental.pallas import tpu as pltpu
from jax.experimental.pallas import tpu_sc as plsc

N = 10000
E = 320000
IN_DIM = 128
OUT_DIM = 16
H = 8
HD = H * OUT_DIM
MROW = HD + 16
NC, NS = 2, 16
NW = NC * NS
EPT = E // NW
C = 40
NCHUNK = EPT // C
RPS = N // NS
ZR = 125
ROW_BLK = 1000


def _proj_body(h_ref, wq_ref, bq_ref, wk_ref, bk_ref, wv_ref, bv_ref,
               q_ref, k_ref, v_ref):
    hb = h_ref[...]
    dn = (((1,), (1,)), ((), ()))
    q_ref[...] = (lax.dot_general(hb, wq_ref[...], dn,
                                  preferred_element_type=jnp.float32)
                  + bq_ref[...]) * 0.25
    k_ref[...] = lax.dot_general(hb, wk_ref[...], dn,
                                 preferred_element_type=jnp.float32) + bk_ref[...]
    v_ref[...] = lax.dot_general(hb, wv_ref[...], dn,
                                 preferred_element_type=jnp.float32) + bv_ref[...]


def _projections(h, Wq, bq, Wk, bk, Wv, bv):
    grid = (N // ROW_BLK,)
    row_spec = pl.BlockSpec((ROW_BLK, IN_DIM), lambda i: (i, 0))
    w_spec = pl.BlockSpec((HD, IN_DIM), lambda i: (0, 0))
    b_spec = pl.BlockSpec((1, HD), lambda i: (0, 0))
    out = jax.ShapeDtypeStruct((N, HD), jnp.float32)
    return pl.pallas_call(
        _proj_body,
        grid=grid,
        in_specs=[row_spec, w_spec, b_spec, w_spec, b_spec, w_spec, b_spec],
        out_specs=[row_spec, row_spec, row_spec],
        out_shape=[out, out, out],
    )(h, Wq, bq.reshape(1, HD), Wk, bk.reshape(1, HD), Wv, bv.reshape(1, HD))


def _edge_body(q_hbm, k_hbm, v_hbm, src_hbm, dst_hbm, zeros_hbm, out_hbm,
               src_v, dst_v, kbuf, qbuf, vbuf, mbuf, acc,
               sem1, sem2, sem3):
    cid = lax.axis_index("c")
    sid = lax.axis_index("s")
    wid = cid * NS + sid

    zeros16 = jnp.zeros((16,), jnp.float32)
    lane = lax.iota(jnp.int32, 16)

    for rep in range(RPS // ZR):
        pltpu.sync_copy(zeros_hbm, acc.at[pl.ds(sid * RPS + rep * ZR, ZR)])
    plsc.subcore_barrier()

    ebase = wid * EPT

    @pl.loop(0, NCHUNK)
    def _(chunk):
        base = ebase + chunk * C
        pltpu.sync_copy(src_hbm.at[pl.ds(base, C)], src_v)
        pltpu.sync_copy(dst_hbm.at[pl.ds(base, C)], dst_v)
        ck = pltpu.async_copy(k_hbm.at[src_v], kbuf, sem1)
        cq = pltpu.async_copy(q_hbm.at[dst_v], qbuf, sem2)
        cv = pltpu.async_copy(v_hbm.at[src_v], vbuf, sem3)
        ck.wait()
        cq.wait()
        cv.wait()

        @pl.loop(0, C)
        def _(e):
            scores = zeros16
            for hh in range(H):
                kk = kbuf[e, pl.ds(hh * 16, 16)]
                qq = qbuf[e, pl.ds(hh * 16, 16)]
                s = jnp.sum(kk * qq)
                sv = jnp.exp(jnp.clip(jnp.full((16,), s, jnp.float32),
                                      -5.0, 5.0))
                vv = vbuf[e, pl.ds(hh * 16, 16)]
                mbuf[e, pl.ds(hh * 16, 16)] = sv * vv
                scores = jnp.where(lane == hh, sv, scores)
            mbuf[e, pl.ds(HD, 16)] = scores

        pltpu.sync_copy(mbuf, acc.at[dst_v], add=True)

    plsc.subcore_barrier()
    pltpu.sync_copy(acc.at[pl.ds(sid * RPS, RPS)],
                    out_hbm.at[cid, pl.ds(sid * RPS, RPS)])


@functools.partial(
    pl.kernel,
    out_type=jax.ShapeDtypeStruct((NC, N, MROW), jnp.float32),
    mesh=plsc.VectorSubcoreMesh(core_axis_name="c", subcore_axis_name="s"),
    scratch_types=[
        pltpu.VMEM((C,), jnp.int32),
        pltpu.VMEM((C,), jnp.int32),
        pltpu.VMEM((C, HD), jnp.float32),
        pltpu.VMEM((C, HD), jnp.float32),
        pltpu.VMEM((C, HD), jnp.float32),
        pltpu.VMEM((C, MROW), jnp.float32),
        pltpu.VMEM_SHARED((N, MROW), jnp.float32),
        pltpu.SemaphoreType.DMA,
        pltpu.SemaphoreType.DMA,
        pltpu.SemaphoreType.DMA,
    ],
    compiler_params=pltpu.CompilerParams(use_tc_tiling_on_sc=False,
                                         needs_layout_passes=False),
)
def _edge_kernel(*refs):
    _edge_body(*refs)


def _merge_body(a_ref, b_ref, o_ref):
    s = a_ref[0] + a_ref[1]
    wv = s[:, :HD]
    z = s[:, HD:HD + H]
    zb = lax.dot_general(z, b_ref[...], (((1,), (0,)), ((), ())),
                         preferred_element_type=jnp.float32)
    o_ref[...] = wv / (zb + 1e-6)


def _merge(parts, bmat):
    grid = (N // ROW_BLK,)
    return pl.pallas_call(
        _merge_body,
        grid=grid,
        in_specs=[pl.BlockSpec((NC, ROW_BLK, MROW), lambda i: (0, i, 0)),
                  pl.BlockSpec((H, HD), lambda i: (0, 0))],
        out_specs=pl.BlockSpec((ROW_BLK, HD), lambda i: (i, 0)),
        out_shape=jax.ShapeDtypeStruct((N, HD), jnp.float32),
    )(parts, bmat)


def kernel(h, edge_index, Wq, bq, Wk, bk, Wv, bv):
    q, k, v = _projections(h, Wq, bq, Wk, bk, Wv, bv)
    src = edge_index[0]
    dst = edge_index[1]
    zeros = jnp.zeros((ZR, MROW), jnp.float32)
    parts = _edge_kernel(q, k, v, src, dst, zeros)
    bmat = (jnp.arange(HD, dtype=jnp.int32) // OUT_DIM
            == jnp.arange(H, dtype=jnp.int32)[:, None]).astype(jnp.float32)
    return _merge(parts, bmat)


# speedup vs baseline: 112.1621x; 7.1240x over previous
import functools

import jax
import jax.numpy as jnp
from jax import lax
from jax.experimental import pallas as pl
from jax.experimental.pallas import tpu as pltpu
from jax.experimental.pallas import tpu_sc as plsc

N = 10000
E = 320000
IN_DIM = 128
OUT_DIM = 16
H = 8
HD = H * OUT_DIM
MROW = HD + 16
NC, NS = 2, 16
NW = NC * NS
EPT = E // NW
C = 40
NCHUNK = EPT // C
CPB = 50
NBLK = NCHUNK // CPB
RPS = N // NS
ZR = 125
ROW_BLK = 1000
PROJ_BLK = 2000


def _proj_body(h_ref, wq_ref, bq_ref, wk_ref, bk_ref, wv_ref, bv_ref,
               qb_ref, kv_ref):
    hb = h_ref[...]
    dn = (((1,), (1,)), ((), ()))
    qb_ref[...] = ((lax.dot_general(hb, wq_ref[...], dn,
                                    preferred_element_type=jnp.float32)
                    + bq_ref[...]) * 0.25).astype(jnp.bfloat16)
    kv_ref[:, :HD] = (lax.dot_general(hb, wk_ref[...], dn,
                                      preferred_element_type=jnp.float32)
                      + bk_ref[...]).astype(jnp.bfloat16)
    kv_ref[:, HD:] = (lax.dot_general(hb, wv_ref[...], dn,
                                      preferred_element_type=jnp.float32)
                      + bv_ref[...]).astype(jnp.bfloat16)


def _projections(h, Wq, bq, Wk, bk, Wv, bv):
    grid = (N // PROJ_BLK,)
    row_spec = pl.BlockSpec((PROJ_BLK, IN_DIM), lambda i: (i, 0))
    w_spec = pl.BlockSpec((HD, IN_DIM), lambda i: (0, 0))
    b_spec = pl.BlockSpec((1, HD), lambda i: (0, 0))
    return pl.pallas_call(
        _proj_body,
        grid=grid,
        in_specs=[row_spec, w_spec, b_spec, w_spec, b_spec, w_spec, b_spec],
        out_specs=[row_spec, pl.BlockSpec((PROJ_BLK, 2 * HD), lambda i: (i, 0))],
        out_shape=[jax.ShapeDtypeStruct((N, HD), jnp.bfloat16),
                   jax.ShapeDtypeStruct((N, 2 * HD), jnp.bfloat16)],
    )(h, Wq, bq.reshape(1, HD), Wk, bk.reshape(1, HD), Wv, bv.reshape(1, HD))


_GDN = lax.GatherDimensionNumbers(offset_dims=(), collapsed_slice_dims=(0,),
                                  start_index_map=(0,))


def _perm(v, idx):
    return lax.gather(v, idx.reshape(16, 1), _GDN, (1,),
                      mode=lax.GatherScatterMode.PROMISE_IN_BOUNDS)


def _edge_compute(kvbuf, qpbuf, mbuf, sbuf):
    lane = lax.iota(jnp.int32, 16)
    masks = [(lane & b) == 0 for b in (8, 4, 2)]
    xors = [lane ^ b for b in (8, 4, 2, 1)]
    reorder = (((lane & 1) << 3) | ((lane & 2) << 1) | ((lane & 4) >> 1)
               | (lane & 8))
    POS = (0, 8, 4, 12, 2, 10, 6, 14)
    pos_idx = [jnp.full((16,), p, jnp.int32) for p in POS]

    def comb(a, b, lvl):
        m, x = masks[lvl], xors[lvl]
        return (jnp.where(m, a, _perm(b, x))
                + jnp.where(m, _perm(a, x), b))

    def unpack2(buf, e, col):
        return plsc.unpack(buf[e, pl.ds(col, 32)],
                           format=plsc.PackFormat.INTERLEAVED,
                           preferred_element_type=jnp.float32)

    @plsc.parallel_loop(0, C, unroll=2)
    def _(e):
        ks, qs = [], []
        for p in range(H // 2):
            ks.extend(unpack2(kvbuf, e, p * 32))
            qs.extend(unpack2(qpbuf, e, p * 32))
        ps = [k * q for k, q in zip(ks, qs)]
        u = [comb(ps[i], ps[i + 1], 0) for i in (0, 2, 4, 6)]
        v = [comb(u[0], u[1], 1), comb(u[2], u[3], 1)]
        w = comb(v[0], v[1], 2)
        s = w + _perm(w, xors[3])
        es = jnp.exp(jnp.clip(s, -5.0, 5.0))
        sbuf[e, pl.ds(0, 16)] = _perm(es, reorder)
        for p in range(H // 2):
            va, vb = unpack2(kvbuf, e, HD + p * 32)
            mbuf[e, pl.ds((2 * p) * 16, 16)] = _perm(es, pos_idx[2 * p]) * va
            mbuf[e, pl.ds((2 * p + 1) * 16, 16)] = (
                _perm(es, pos_idx[2 * p + 1]) * vb)


def _edge_body(q_hbm, kv_hbm, ei_hbm, zw_hbm, zs_hbm,
               outw_hbm, outs_hbm,
               src_blk, dst_blk, kvb0, kvb1, qb0, qb1, mb0, mb1, sb0, sb1,
               accw, accs,
               semk0, semk1, semq0, semq1, semw0, semw1, sems0, sems1):
    cid = lax.axis_index("c")
    sid = lax.axis_index("s")
    wid = cid * NS + sid

    kvbuf = (kvb0, kvb1)
    qpbuf = (qb0, qb1)
    mbuf = (mb0, mb1)
    sbuf = (sb0, sb1)
    semk = (semk0, semk1)
    semq = (semq0, semq1)
    semw = (semw0, semw1)
    sems = (sems0, sems1)

    for rep in range(RPS // ZR):
        pltpu.sync_copy(zw_hbm, accw.at[pl.ds(sid * RPS + rep * ZR, ZR)])
        pltpu.sync_copy(zs_hbm, accs.at[pl.ds(sid * RPS + rep * ZR, ZR)])
    plsc.subcore_barrier()

    def gathers(j, slot):
        pltpu.async_copy(kv_hbm.at[src_blk.at[j]], kvbuf[slot], semk[slot])
        pltpu.async_copy(q_hbm.at[dst_blk.at[j]], qpbuf[slot], semq[slot])

    def wait_gathers(j, slot):
        pltpu.make_async_copy(kv_hbm.at[src_blk.at[j]], kvbuf[slot],
                              semk[slot]).wait()
        pltpu.make_async_copy(q_hbm.at[dst_blk.at[j]], qpbuf[slot],
                              semq[slot]).wait()

    def scatters(j, slot):
        pltpu.async_copy(mbuf[slot], accw.at[dst_blk.at[j]], semw[slot],
                         add=True)
        pltpu.async_copy(sbuf[slot], accs.at[dst_blk.at[j]], sems[slot],
                         add=True)

    def wait_scatters(slot):
        pltpu.make_async_copy(mbuf[slot], accw.at[dst_blk.at[0]],
                              semw[slot]).wait()
        pltpu.make_async_copy(sbuf[slot], accs.at[dst_blk.at[0]],
                              sems[slot]).wait()

    row0 = wid * NCHUNK
    for blk in range(NBLK):
        if blk > 0:
            wait_scatters(0)
            wait_scatters(1)
        pltpu.sync_copy(ei_hbm.at[0, pl.ds(row0 + blk * CPB, CPB)], src_blk)
        pltpu.sync_copy(ei_hbm.at[1, pl.ds(row0 + blk * CPB, CPB)], dst_blk)
        gathers(0, 0)

        @pl.loop(0, CPB, step=2)
        def _(chunk):
            for par in range(2):
                cur = par
                j = chunk + par

                @pl.when(j + 1 < CPB)
                def _():
                    gathers(j + 1, 1 - par)

                wait_gathers(j, cur)

                @pl.when(j >= 2)
                def _():
                    wait_scatters(cur)

                _edge_compute(kvbuf[cur], qpbuf[cur], mbuf[cur], sbuf[cur])
                scatters(j, cur)

    wait_scatters(0)
    wait_scatters(1)
    plsc.subcore_barrier()
    pltpu.sync_copy(accw.at[pl.ds(sid * RPS, RPS)],
                    outw_hbm.at[cid, pl.ds(sid * RPS, RPS)])
    pltpu.sync_copy(accs.at[pl.ds(sid * RPS, RPS)],
                    outs_hbm.at[cid, pl.ds(sid * RPS, RPS)])


@functools.partial(
    pl.kernel,
    out_type=[jax.ShapeDtypeStruct((NC, N, HD), jnp.float32),
              jax.ShapeDtypeStruct((NC, N, 16), jnp.float32)],
    mesh=plsc.VectorSubcoreMesh(core_axis_name="c", subcore_axis_name="s"),
    scratch_types=[
        pltpu.VMEM((CPB, C), jnp.int32),
        pltpu.VMEM((CPB, C), jnp.int32),
        pltpu.VMEM((C, 2 * HD), jnp.bfloat16),
        pltpu.VMEM((C, 2 * HD), jnp.bfloat16),
        pltpu.VMEM((C, HD), jnp.bfloat16),
        pltpu.VMEM((C, HD), jnp.bfloat16),
        pltpu.VMEM((C, HD), jnp.float32),
        pltpu.VMEM((C, HD), jnp.float32),
        pltpu.VMEM((C, 16), jnp.float32),
        pltpu.VMEM((C, 16), jnp.float32),
        pltpu.VMEM_SHARED((N, HD), jnp.float32),
        pltpu.VMEM_SHARED((N, 16), jnp.float32),
        pltpu.SemaphoreType.DMA,
        pltpu.SemaphoreType.DMA,
        pltpu.SemaphoreType.DMA,
        pltpu.SemaphoreType.DMA,
        pltpu.SemaphoreType.DMA,
        pltpu.SemaphoreType.DMA,
        pltpu.SemaphoreType.DMA,
        pltpu.SemaphoreType.DMA,
    ],
    compiler_params=pltpu.CompilerParams(use_tc_tiling_on_sc=False,
                                         needs_layout_passes=False),
)
def _edge_kernel(*refs):
    _edge_body(*refs)


def _merge_body(aw_ref, as_ref, b_ref, o_ref):
    wv = aw_ref[0] + aw_ref[1]
    z = (as_ref[0] + as_ref[1])[:, :H]
    zb = lax.dot_general(z, b_ref[...], (((1,), (0,)), ((), ())),
                         preferred_element_type=jnp.float32)
    o_ref[...] = wv / (zb + 1e-6)


def _merge(parts_w, parts_s, bmat):
    grid = (N // ROW_BLK,)
    return pl.pallas_call(
        _merge_body,
        grid=grid,
        in_specs=[pl.BlockSpec((NC, ROW_BLK, HD), lambda i: (0, i, 0)),
                  pl.BlockSpec((NC, ROW_BLK, 16), lambda i: (0, i, 0)),
                  pl.BlockSpec((H, HD), lambda i: (0, 0))],
        out_specs=pl.BlockSpec((ROW_BLK, HD), lambda i: (i, 0)),
        out_shape=jax.ShapeDtypeStruct((N, HD), jnp.float32),
    )(parts_w, parts_s, bmat)


def kernel(h, edge_index, Wq, bq, Wk, bk, Wv, bv):
    p_idx = jnp.arange(HD, dtype=jnp.int32)
    colmap = (2 * (p_idx // 32) + (p_idx % 2)) * 16 + (p_idx % 32) // 2
    qb, kv = _projections(h, Wq[colmap], bq[colmap], Wk[colmap], bk[colmap],
                          Wv[colmap], bv[colmap])
    ei3 = edge_index.reshape(2, E // C, C)
    zeros_w = jnp.zeros((ZR, HD), jnp.float32)
    zeros_s = jnp.zeros((ZR, 16), jnp.float32)
    parts_w, parts_s = _edge_kernel(qb, kv, ei3, zeros_w, zeros_s)
    bmat = (jnp.arange(HD, dtype=jnp.int32) // OUT_DIM
            == jnp.arange(H, dtype=jnp.int32)[:, None]).astype(jnp.float32)
    return _merge(parts_w, parts_s, bmat)
